# baseline (speedup 1.0000x reference)
import functools

import jax
import jax.numpy as jnp
from jax.experimental import pallas as pl
from jax.experimental.pallas import tpu as pltpu


def _cparams(sem):
    return pltpu.CompilerParams(dimension_semantics=sem,
                                vmem_limit_bytes=64 * 1024 * 1024)


def _xcol(a):
    Hi, W, C = a.shape
    zc = jnp.zeros((Hi, 1, C), a.dtype)
    left = jnp.concatenate([zc, a[:, :W - 1, :]], axis=1)
    right = jnp.concatenate([a[:, 1:, :], zc], axis=1)
    return jnp.concatenate([left, a, right], axis=-1)


def _conv_relu(a, w_ref, b_ref):
    hi, W, cin = a.shape
    Ho = hi - 2
    cout = w_ref.shape[-1]
    xc = _xcol(a)
    acc = None
    for dy in range(3):
        av = xc[dy:dy + Ho].reshape(Ho * W, 3 * cin)
        wk = w_ref[3 * dy:3 * dy + 3].reshape(3 * cin, cout)
        p = jnp.dot(av, wk, preferred_element_type=jnp.float32)
        acc = p if acc is None else acc + p
    return jnp.maximum(acc + b_ref[...], 0.0)


def _conv_relu_npack(a, wc_ref, b_ref, cout):
    hi, W, cin = a.shape
    Ho = hi - 2
    G = wc_ref.shape[-1] // 3
    xc = _xcol(a).reshape(hi * W, 3 * cin)
    P = jnp.dot(xc, wc_ref[...], preferred_element_type=jnp.float32)
    acc = None
    for dy in range(3):
        s = P[dy * W:dy * W + Ho * W, dy * G:dy * G + cout]
        acc = s if acc is None else acc + s
    return jnp.maximum(acc + b_ref[...], 0.0)


def _npack_w(w9):
    _, cin, cout = w9.shape
    g = max(cout, 128)
    wc = w9.reshape(3, 3, cin, cout).transpose(1, 2, 0, 3)
    if g != cout:
        wc = jnp.pad(wc, ((0, 0), (0, 0), (0, 0), (0, g - cout)))
    return wc.reshape(3 * cin, 3 * g)


def _pool_flat(r, H, W, cout):
    r = r.reshape(H // 2, 2, W, cout)
    r = jnp.maximum(r[:, 0], r[:, 1])
    r = r.reshape(H // 2, W // 2, 2, cout)
    return jnp.maximum(r[:, :, 0, :], r[:, :, 1, :])


def _dual_body(top_ref, x_ref, bot_ref, w1, b1, w2, b2, o_ref, *, npack):
    th, W, cin = x_ref.shape
    c2 = o_ref.shape[-1]
    c1 = (w2.shape[0] // 3) if npack else w2.shape[1]
    i = pl.program_id(1)
    nb = pl.num_programs(1)

    top = top_ref[...]
    top = jnp.where(i == 0, jnp.zeros_like(top), top)
    bot = bot_ref[...]
    bot = jnp.where(i == nb - 1, jnp.zeros_like(bot), bot)
    xin = jnp.concatenate([top, x_ref[...], bot], axis=0)

    if npack:
        r1 = _conv_relu_npack(xin, w1, b1, c1)
    else:
        r1 = _conv_relu(xin, w1, b1)
    a1 = r1.reshape(th + 2, W, c1).astype(jnp.bfloat16)
    row0 = jnp.where(i == 0, jnp.zeros_like(a1[0:1]), a1[0:1])
    rowl = jnp.where(i == nb - 1, jnp.zeros_like(a1[0:1]), a1[th + 1:th + 2])
    a1 = jnp.concatenate([row0, a1[1:th + 1], rowl], axis=0)

    if npack:
        r2 = _conv_relu_npack(a1, w2, b2, c2)
    else:
        r2 = _conv_relu(a1, w2, b2)
    o_ref[...] = _pool_flat(r2, th, W, c2).astype(o_ref.dtype)


def _dual_block(x, w1, b1, w2, b2, *, th, npack=False):
    N, H, W, cin = x.shape
    c1, c2 = w1.shape[-1], w2.shape[-1]
    assert H % th == 0 and th % 2 == 0
    hh = th // 2
    if npack:
        wa, wb = _npack_w(w1), _npack_w(w2)
        wspecs = [pl.BlockSpec(wa.shape, lambda n, i: (0, 0)),
                  pl.BlockSpec((1, c1), lambda n, i: (0, 0)),
                  pl.BlockSpec(wb.shape, lambda n, i: (0, 0)),
                  pl.BlockSpec((1, c2), lambda n, i: (0, 0))]
    else:
        wa, wb = w1, w2
        wspecs = [pl.BlockSpec((9, cin, c1), lambda n, i: (0, 0, 0)),
                  pl.BlockSpec((1, c1), lambda n, i: (0, 0)),
                  pl.BlockSpec((9, c1, c2), lambda n, i: (0, 0, 0)),
                  pl.BlockSpec((1, c2), lambda n, i: (0, 0))]

    return pl.pallas_call(
        functools.partial(_dual_body, npack=npack),
        out_shape=jax.ShapeDtypeStruct((N, H // 2, W // 2, c2), jnp.bfloat16),
        grid_spec=pltpu.PrefetchScalarGridSpec(
            num_scalar_prefetch=0,
            grid=(N, H // th),
            in_specs=[
                pl.BlockSpec((None, 2, W, cin),
                             lambda n, i: (n, jnp.maximum(i * hh - 1, 0), 0, 0)),
                pl.BlockSpec((None, th, W, cin), lambda n, i: (n, i, 0, 0)),
                pl.BlockSpec((None, 2, W, cin),
                             lambda n, i: (n, jnp.minimum((i + 1) * hh, H // 2 - 1), 0, 0)),
            ] + wspecs,
            out_specs=pl.BlockSpec((None, th // 2, W // 2, c2),
                                   lambda n, i: (n, i, 0, 0))),
        compiler_params=_cparams(("parallel", "parallel")),
    )(x, x, x, wa, b1.reshape(1, c1), wb, b2.reshape(1, c2))


def _tri_body(x_ref, w1, b1, w2, b2, w3, b3, o_ref):
    nimg, H, W, cin = x_ref.shape
    c3 = w3.shape[-1]
    z = jnp.zeros((1, W, cin), jnp.bfloat16)
    parts = []
    for m in range(nimg):
        parts += [z, x_ref[m], z]
    a = jnp.concatenate(parts, axis=0)
    for w, b in ((w1, b1), (w2, b2)):
        c = w.shape[-1]
        r = _conv_relu(a, w, b)
        rb = r.reshape(nimg * (H + 2) - 2, W, c).astype(jnp.bfloat16)
        zc = jnp.zeros((1, W, c), jnp.bfloat16)
        parts = []
        for m in range(nimg):
            parts += [zc, rb[m * (H + 2):m * (H + 2) + H], zc]
        a = jnp.concatenate(parts, axis=0)
    r = _conv_relu(a, w3, b3)
    for m in range(nimg):
        band = r[m * (H + 2) * W:(m * (H + 2) + H) * W]
        o_ref[m] = _pool_flat(band, H, W, c3).astype(o_ref.dtype)


def _tri_block(x, w1, b1, w2, b2, w3, b3, *, nimg=2):
    N, H, W, cin = x.shape
    c1, c2, c3 = w1.shape[-1], w2.shape[-1], w3.shape[-1]
    assert N % nimg == 0

    return pl.pallas_call(
        _tri_body,
        out_shape=jax.ShapeDtypeStruct((N, H // 2, W // 2, c3), jnp.bfloat16),
        grid_spec=pltpu.PrefetchScalarGridSpec(
            num_scalar_prefetch=0,
            grid=(N // nimg,),
            in_specs=[
                pl.BlockSpec((nimg, H, W, cin), lambda n: (n, 0, 0, 0)),
                pl.BlockSpec((9, cin, c1), lambda n: (0, 0, 0)),
                pl.BlockSpec((1, c1), lambda n: (0, 0)),
                pl.BlockSpec((9, c1, c2), lambda n: (0, 0, 0)),
                pl.BlockSpec((1, c2), lambda n: (0, 0)),
                pl.BlockSpec((9, c2, c3), lambda n: (0, 0, 0)),
                pl.BlockSpec((1, c3), lambda n: (0, 0)),
            ],
            out_specs=pl.BlockSpec((nimg, H // 2, W // 2, c3),
                                   lambda n: (n, 0, 0, 0))),
        compiler_params=_cparams(("parallel",)),
    )(x, w1, b1.reshape(1, c1), w2, b2.reshape(1, c2), w3, b3.reshape(1, c3))


def _fc_body(a_ref, b_ref, bias_ref, o_ref, acc_ref):
    k = pl.program_id(1)

    @pl.when(k == 0)
    def _():
        acc_ref[...] = jnp.zeros_like(acc_ref)

    acc_ref[...] += jnp.dot(a_ref[...], b_ref[...],
                            preferred_element_type=jnp.float32)

    @pl.when(k == pl.num_programs(1) - 1)
    def _():
        o_ref[...] = acc_ref[...] + bias_ref[...]


def _fc(a, w, bias, *, tn, tk):
    M, K = a.shape
    _, Nc = w.shape
    return pl.pallas_call(
        _fc_body,
        out_shape=jax.ShapeDtypeStruct((M, Nc), jnp.float32),
        grid_spec=pltpu.PrefetchScalarGridSpec(
            num_scalar_prefetch=0,
            grid=(Nc // tn, K // tk),
            in_specs=[pl.BlockSpec((M, tk), lambda j, k: (0, k)),
                      pl.BlockSpec((tk, tn), lambda j, k: (k, j)),
                      pl.BlockSpec((1, tn), lambda j, k: (0, j))],
            out_specs=pl.BlockSpec((M, tn), lambda j, k: (0, j)),
            scratch_shapes=[pltpu.VMEM((M, tn), jnp.float32)]),
        compiler_params=_cparams(("parallel", "arbitrary")),
    )(a, w, bias.reshape(1, Nc))


def kernel(x, conv_1_1_w, conv_1_1_b, conv_1_2_w, conv_1_2_b,
           conv_2_1_w, conv_2_1_b, conv_2_2_w, conv_2_2_b,
           conv_3_1_w, conv_3_1_b, conv_3_2_w, conv_3_2_b,
           conv_3_3_w, conv_3_3_b, conv_4_1_w, conv_4_1_b,
           conv_4_2_w, conv_4_2_b, conv_4_3_w, conv_4_3_b,
           conv_5_1_w, conv_5_1_b, conv_5_2_w, conv_5_2_b,
           conv_5_3_w, conv_5_3_b, fc6_w, fc6_b):
    h = jnp.transpose(x, (0, 2, 3, 1)).astype(jnp.bfloat16)
    h = _dual_block(h, conv_1_1_w, conv_1_1_b, conv_1_2_w, conv_1_2_b, th=56)
    h = _dual_block(h, conv_2_1_w, conv_2_1_b, conv_2_2_w, conv_2_2_b,
                    th=56, npack=True)
    h = _tri_block(h, conv_3_1_w, conv_3_1_b, conv_3_2_w, conv_3_2_b,
                   conv_3_3_w, conv_3_3_b)
    h = _tri_block(h, conv_4_1_w, conv_4_1_b, conv_4_2_w, conv_4_2_b,
                   conv_4_3_w, conv_4_3_b)
    h = _tri_block(h, conv_5_1_w, conv_5_1_b, conv_5_2_w, conv_5_2_b,
                   conv_5_3_w, conv_5_3_b)
    n = h.shape[0]
    flat = jnp.transpose(h, (0, 3, 1, 2)).reshape(n, -1)
    return _fc(flat, fc6_w, fc6_b, tn=1024, tk=3584)


# speedup vs baseline: 1.0845x; 1.0845x over previous
import functools

import jax
import jax.numpy as jnp
from jax.experimental import pallas as pl
from jax.experimental.pallas import tpu as pltpu


def _cparams(sem):
    return pltpu.CompilerParams(dimension_semantics=sem,
                                vmem_limit_bytes=64 * 1024 * 1024)


def _xcol(a):
    Hi, W, C = a.shape
    zc = jnp.zeros((Hi, 1, C), a.dtype)
    left = jnp.concatenate([zc, a[:, :W - 1, :]], axis=1)
    right = jnp.concatenate([a[:, 1:, :], zc], axis=1)
    return jnp.concatenate([left, a, right], axis=-1)


def _conv_relu(a, w_ref, b_ref):
    hi, W, cin = a.shape
    Ho = hi - 2
    cout = w_ref.shape[-1]
    xc = _xcol(a)
    acc = None
    for dy in range(3):
        av = xc[dy:dy + Ho].reshape(Ho * W, 3 * cin)
        wk = w_ref[3 * dy:3 * dy + 3].reshape(3 * cin, cout)
        p = jnp.dot(av, wk, preferred_element_type=jnp.float32)
        acc = p if acc is None else acc + p
    return jnp.maximum(acc + b_ref[...], 0.0)


def _conv_relu_npack(a, wc_ref, b_ref, cout):
    hi, W, cin = a.shape
    Ho = hi - 2
    G = wc_ref.shape[-1] // 3
    xc = _xcol(a).reshape(hi * W, 3 * cin)
    P = jnp.dot(xc, wc_ref[...], preferred_element_type=jnp.float32)
    acc = None
    for dy in range(3):
        s = P[dy * W:dy * W + Ho * W, dy * G:dy * G + cout]
        acc = s if acc is None else acc + s
    return jnp.maximum(acc + b_ref[...], 0.0)


def _npack_w(w9):
    _, cin, cout = w9.shape
    g = max(cout, 128)
    wc = w9.reshape(3, 3, cin, cout).transpose(1, 2, 0, 3)
    if g != cout:
        wc = jnp.pad(wc, ((0, 0), (0, 0), (0, 0), (0, g - cout)))
    return wc.reshape(3 * cin, 3 * g)


def _pool_flat(r, H, W, cout):
    r = r.reshape(H // 2, 2, W, cout)
    r = jnp.maximum(r[:, 0], r[:, 1])
    r = r.reshape(H // 2, W // 2, 2, cout)
    return jnp.maximum(r[:, :, 0, :], r[:, :, 1, :])


def _dual_body(top_ref, x_ref, bot_ref, w1, b1, w2, b2, o_ref, *, npack):
    th, W, cin = x_ref.shape
    c2 = o_ref.shape[-1]
    c1 = (w2.shape[0] // 3) if npack else w2.shape[1]
    i = pl.program_id(1)
    nb = pl.num_programs(1)

    top = top_ref[...]
    top = jnp.where(i == 0, jnp.zeros_like(top), top)
    bot = bot_ref[...]
    bot = jnp.where(i == nb - 1, jnp.zeros_like(bot), bot)
    xin = jnp.concatenate([top, x_ref[...], bot], axis=0)

    if npack:
        r1 = _conv_relu_npack(xin, w1, b1, c1)
    else:
        r1 = _conv_relu(xin, w1, b1)
    a1 = r1.reshape(th + 2, W, c1).astype(jnp.bfloat16)
    row0 = jnp.where(i == 0, jnp.zeros_like(a1[0:1]), a1[0:1])
    rowl = jnp.where(i == nb - 1, jnp.zeros_like(a1[0:1]), a1[th + 1:th + 2])
    a1 = jnp.concatenate([row0, a1[1:th + 1], rowl], axis=0)

    if npack:
        r2 = _conv_relu_npack(a1, w2, b2, c2)
    else:
        r2 = _conv_relu(a1, w2, b2)
    o_ref[...] = _pool_flat(r2, th, W, c2).astype(o_ref.dtype)


def _dual_block(x, w1, b1, w2, b2, *, th, npack=False):
    N, H, W, cin = x.shape
    c1, c2 = w1.shape[-1], w2.shape[-1]
    assert H % th == 0 and th % 2 == 0
    hh = th // 2
    if npack:
        wa, wb = _npack_w(w1), _npack_w(w2)
        wspecs = [pl.BlockSpec(wa.shape, lambda n, i: (0, 0)),
                  pl.BlockSpec((1, c1), lambda n, i: (0, 0)),
                  pl.BlockSpec(wb.shape, lambda n, i: (0, 0)),
                  pl.BlockSpec((1, c2), lambda n, i: (0, 0))]
    else:
        wa, wb = w1, w2
        wspecs = [pl.BlockSpec((9, cin, c1), lambda n, i: (0, 0, 0)),
                  pl.BlockSpec((1, c1), lambda n, i: (0, 0)),
                  pl.BlockSpec((9, c1, c2), lambda n, i: (0, 0, 0)),
                  pl.BlockSpec((1, c2), lambda n, i: (0, 0))]

    return pl.pallas_call(
        functools.partial(_dual_body, npack=npack),
        out_shape=jax.ShapeDtypeStruct((N, H // 2, W // 2, c2), jnp.bfloat16),
        grid_spec=pltpu.PrefetchScalarGridSpec(
            num_scalar_prefetch=0,
            grid=(N, H // th),
            in_specs=[
                pl.BlockSpec((None, 2, W, cin),
                             lambda n, i: (n, jnp.maximum(i * hh - 1, 0), 0, 0)),
                pl.BlockSpec((None, th, W, cin), lambda n, i: (n, i, 0, 0)),
                pl.BlockSpec((None, 2, W, cin),
                             lambda n, i: (n, jnp.minimum((i + 1) * hh, H // 2 - 1), 0, 0)),
            ] + wspecs,
            out_specs=pl.BlockSpec((None, th // 2, W // 2, c2),
                                   lambda n, i: (n, i, 0, 0))),
        compiler_params=_cparams(("parallel", "parallel")),
    )(x, x, x, wa, b1.reshape(1, c1), wb, b2.reshape(1, c2))


def _tri_body(x_ref, w1, b1, w2, b2, w3, b3, o_ref):
    nimg, H, W, cin = x_ref.shape
    c3 = w3.shape[-1]
    z = jnp.zeros((1, W, cin), jnp.bfloat16)
    parts = []
    for m in range(nimg):
        parts += [z, x_ref[m], z]
    a = jnp.concatenate(parts, axis=0)
    for w, b in ((w1, b1), (w2, b2)):
        c = w.shape[-1]
        r = _conv_relu(a, w, b)
        rb = r.reshape(nimg * (H + 2) - 2, W, c).astype(jnp.bfloat16)
        zc = jnp.zeros((1, W, c), jnp.bfloat16)
        parts = []
        for m in range(nimg):
            parts += [zc, rb[m * (H + 2):m * (H + 2) + H], zc]
        a = jnp.concatenate(parts, axis=0)
    r = _conv_relu(a, w3, b3)
    for m in range(nimg):
        band = r[m * (H + 2) * W:(m * (H + 2) + H) * W]
        o_ref[m] = _pool_flat(band, H, W, c3).astype(o_ref.dtype)


def _tri_block(x, w1, b1, w2, b2, w3, b3, *, nimg=1):
    N, H, W, cin = x.shape
    c1, c2, c3 = w1.shape[-1], w2.shape[-1], w3.shape[-1]
    assert N % nimg == 0

    return pl.pallas_call(
        _tri_body,
        out_shape=jax.ShapeDtypeStruct((N, H // 2, W // 2, c3), jnp.bfloat16),
        grid_spec=pltpu.PrefetchScalarGridSpec(
            num_scalar_prefetch=0,
            grid=(N // nimg,),
            in_specs=[
                pl.BlockSpec((nimg, H, W, cin), lambda n: (n, 0, 0, 0)),
                pl.BlockSpec((9, cin, c1), lambda n: (0, 0, 0)),
                pl.BlockSpec((1, c1), lambda n: (0, 0)),
                pl.BlockSpec((9, c1, c2), lambda n: (0, 0, 0)),
                pl.BlockSpec((1, c2), lambda n: (0, 0)),
                pl.BlockSpec((9, c2, c3), lambda n: (0, 0, 0)),
                pl.BlockSpec((1, c3), lambda n: (0, 0)),
            ],
            out_specs=pl.BlockSpec((nimg, H // 2, W // 2, c3),
                                   lambda n: (n, 0, 0, 0))),
        compiler_params=_cparams(("parallel",)),
    )(x, w1, b1.reshape(1, c1), w2, b2.reshape(1, c2), w3, b3.reshape(1, c3))


def _fc_body(a_ref, b_ref, bias_ref, o_ref, acc_ref):
    k = pl.program_id(1)

    @pl.when(k == 0)
    def _():
        acc_ref[...] = jnp.zeros_like(acc_ref)

    acc_ref[...] += jnp.dot(a_ref[...], b_ref[...],
                            preferred_element_type=jnp.float32)

    @pl.when(k == pl.num_programs(1) - 1)
    def _():
        o_ref[...] = acc_ref[...] + bias_ref[...]


def _fc(a, w, bias, *, tn, tk):
    M, K = a.shape
    _, Nc = w.shape
    return pl.pallas_call(
        _fc_body,
        out_shape=jax.ShapeDtypeStruct((M, Nc), jnp.float32),
        grid_spec=pltpu.PrefetchScalarGridSpec(
            num_scalar_prefetch=0,
            grid=(Nc // tn, K // tk),
            in_specs=[pl.BlockSpec((M, tk), lambda j, k: (0, k)),
                      pl.BlockSpec((tk, tn), lambda j, k: (k, j)),
                      pl.BlockSpec((1, tn), lambda j, k: (0, j))],
            out_specs=pl.BlockSpec((M, tn), lambda j, k: (0, j)),
            scratch_shapes=[pltpu.VMEM((M, tn), jnp.float32)]),
        compiler_params=_cparams(("parallel", "arbitrary")),
    )(a, w, bias.reshape(1, Nc))


def kernel(x, conv_1_1_w, conv_1_1_b, conv_1_2_w, conv_1_2_b,
           conv_2_1_w, conv_2_1_b, conv_2_2_w, conv_2_2_b,
           conv_3_1_w, conv_3_1_b, conv_3_2_w, conv_3_2_b,
           conv_3_3_w, conv_3_3_b, conv_4_1_w, conv_4_1_b,
           conv_4_2_w, conv_4_2_b, conv_4_3_w, conv_4_3_b,
           conv_5_1_w, conv_5_1_b, conv_5_2_w, conv_5_2_b,
           conv_5_3_w, conv_5_3_b, fc6_w, fc6_b):
    h = jnp.transpose(x, (0, 2, 3, 1)).astype(jnp.bfloat16)
    h = _dual_block(h, conv_1_1_w, conv_1_1_b, conv_1_2_w, conv_1_2_b,
                    th=56, npack=True)
    h = _dual_block(h, conv_2_1_w, conv_2_1_b, conv_2_2_w, conv_2_2_b,
                    th=56, npack=True)
    h = _tri_block(h, conv_3_1_w, conv_3_1_b, conv_3_2_w, conv_3_2_b,
                   conv_3_3_w, conv_3_3_b)
    h = _tri_block(h, conv_4_1_w, conv_4_1_b, conv_4_2_w, conv_4_2_b,
                   conv_4_3_w, conv_4_3_b)
    h = _tri_block(h, conv_5_1_w, conv_5_1_b, conv_5_2_w, conv_5_2_b,
                   conv_5_3_w, conv_5_3_b)
    n = h.shape[0]
    flat = jnp.transpose(h, (0, 3, 1, 2)).reshape(n, -1)
    return _fc(flat, fc6_w, fc6_b, tn=1024, tk=3584)
